# baseline (speedup 1.0000x reference)
import functools

import jax
import jax.numpy as jnp
from jax.experimental import pallas as pl
from jax.experimental.pallas import tpu as pltpu

LANES = 128


def _gelu(v):
    return 0.5 * v * (1.0 + jax.lax.erf(v * (2.0 ** -0.5)))


def _gate_body(x_ref, gw_ref, gb_ref, tv_ref, idx_ref, loss_ref, imp_ref, *, E):
    t = pl.program_id(0)
    logits = jnp.dot(x_ref[...], gw_ref[...], preferred_element_type=jnp.float32)
    logits = logits + gb_ref[...]
    lane = jax.lax.broadcasted_iota(jnp.int32, logits.shape, 1)
    valid = lane < E
    lm = jnp.where(valid, logits, -1e30)
    m = jnp.max(lm, axis=1, keepdims=True)
    p = jnp.where(valid, jnp.exp(lm - m), 0.0)
    p = p / jnp.sum(p, axis=1, keepdims=True)
    tv = jnp.max(p, axis=1, keepdims=True)
    idx = jnp.min(jnp.where(p == tv, lane, LANES), axis=1, keepdims=True)
    tv_ref[...] = tv
    idx_ref[...] = idx

    @pl.when(t == 0)
    def _():
        imp_ref[...] = jnp.zeros_like(imp_ref)

    imp_ref[...] += jnp.sum(p, axis=0, keepdims=True)

    @pl.when(t == pl.num_programs(0) - 1)
    def _():
        imp = imp_ref[...]
        vmask = jax.lax.broadcasted_iota(jnp.int32, imp.shape, 1) < E
        mean = jnp.sum(imp) / E
        var = jnp.sum(jnp.where(vmask, (imp - mean) ** 2, 0.0)) / (E - 1)
        loss_ref[...] = var / (mean * mean + 1e-10) * jnp.ones_like(loss_ref)


def _expert_body(x_ref, w1_ref, b1_ref, w2_ref, b2_ref, tv_ref, idx_ref,
                 g_ref, out_ref, *, E, BT, D):
    e = pl.program_id(0)
    t = pl.program_id(1)
    xb = x_ref[...]
    bf = jnp.bfloat16
    h = jnp.dot(xb.astype(bf), w1_ref[0].astype(bf),
                preferred_element_type=jnp.float32) + b1_ref[0]
    h = _gelu(h)
    o = jnp.dot(h.astype(bf), w2_ref[0].astype(bf),
                preferred_element_type=jnp.float32) + b2_ref[0]
    contrib = jnp.where(idx_ref[...] == e, o, 0.0)
    sl = pl.ds(t * BT, BT)

    @pl.when(e == 0)
    def _():
        out_ref[sl, :] = contrib

    @pl.when(e != 0)
    def _():
        out_ref[sl, :] += contrib

    @pl.when(e == E - 1)
    def _():
        y = xb + out_ref[sl, :] * tv_ref[...]
        nrm = jnp.sqrt(jnp.sum(y * y, axis=1, keepdims=True))
        y_n = y / jnp.maximum(nrm, 1e-12) * g_ref[...] * (D ** 0.5)
        out_ref[sl, :] = _gelu(y_n)


def kernel(x, gate_W, gate_b, W1, b1, W2, b2, gamma):
    B, N, D = x.shape
    E, _, H = W1.shape
    x_flat = x.reshape(N, D)
    BT = 256
    T = N // BT

    gwp = jnp.zeros((D, LANES), jnp.float32).at[:, :E].set(gate_W)
    gbp = jnp.zeros((1, LANES), jnp.float32).at[0, :E].set(gate_b)

    tv, idx, loss = pl.pallas_call(
        functools.partial(_gate_body, E=E),
        grid=(T,),
        in_specs=[
            pl.BlockSpec((BT, D), lambda t: (t, 0)),
            pl.BlockSpec((D, LANES), lambda t: (0, 0)),
            pl.BlockSpec((1, LANES), lambda t: (0, 0)),
        ],
        out_specs=[
            pl.BlockSpec((BT, 1), lambda t: (t, 0)),
            pl.BlockSpec((BT, 1), lambda t: (t, 0)),
            pl.BlockSpec((1, 1), lambda t: (0, 0)),
        ],
        out_shape=[
            jax.ShapeDtypeStruct((N, 1), jnp.float32),
            jax.ShapeDtypeStruct((N, 1), jnp.int32),
            jax.ShapeDtypeStruct((1, 1), jnp.float32),
        ],
        scratch_shapes=[pltpu.VMEM((1, LANES), jnp.float32)],
    )(x_flat, gwp, gbp)

    out = pl.pallas_call(
        functools.partial(_expert_body, E=E, BT=BT, D=D),
        grid=(E, T),
        in_specs=[
            pl.BlockSpec((BT, D), lambda e, t: (t, 0)),
            pl.BlockSpec((1, D, H), lambda e, t: (e, 0, 0)),
            pl.BlockSpec((1, 1, H), lambda e, t: (e, 0, 0)),
            pl.BlockSpec((1, H, D), lambda e, t: (e, 0, 0)),
            pl.BlockSpec((1, 1, D), lambda e, t: (e, 0, 0)),
            pl.BlockSpec((BT, 1), lambda e, t: (t, 0)),
            pl.BlockSpec((BT, 1), lambda e, t: (t, 0)),
            pl.BlockSpec((1, D), lambda e, t: (0, 0)),
        ],
        out_specs=pl.BlockSpec((N, D), lambda e, t: (0, 0)),
        out_shape=jax.ShapeDtypeStruct((N, D), jnp.float32),
    )(x_flat, W1, b1.reshape(E, 1, H), W2, b2.reshape(E, 1, D), tv, idx,
      gamma.reshape(1, D))

    return out.reshape(B, N, D), loss.reshape(())


# speedup vs baseline: 1.0191x; 1.0191x over previous
import functools

import jax
import jax.numpy as jnp
from jax.experimental import pallas as pl
from jax.experimental.pallas import tpu as pltpu

LANES = 128


def _gelu(v):
    return 0.5 * v * (1.0 + jax.lax.erf(v * (2.0 ** -0.5)))


def _gate_body(x_ref, gw_ref, gb_ref, tv_ref, idx_ref, loss_ref, imp_ref, *, E):
    t = pl.program_id(0)
    logits = jnp.dot(x_ref[...], gw_ref[...], preferred_element_type=jnp.float32)
    logits = logits + gb_ref[...]
    lane = jax.lax.broadcasted_iota(jnp.int32, logits.shape, 1)
    valid = lane < E
    lm = jnp.where(valid, logits, -1e30)
    m = jnp.max(lm, axis=1, keepdims=True)
    p = jnp.where(valid, jnp.exp(lm - m), 0.0)
    p = p / jnp.sum(p, axis=1, keepdims=True)
    tv = jnp.max(p, axis=1, keepdims=True)
    idx = jnp.min(jnp.where(p == tv, lane, LANES), axis=1, keepdims=True)
    tv_ref[...] = tv
    idx_ref[...] = idx

    @pl.when(t == 0)
    def _():
        imp_ref[...] = jnp.zeros_like(imp_ref)

    imp_ref[...] += jnp.sum(p, axis=0, keepdims=True)

    @pl.when(t == pl.num_programs(0) - 1)
    def _():
        imp = imp_ref[...]
        vmask = jax.lax.broadcasted_iota(jnp.int32, imp.shape, 1) < E
        mean = jnp.sum(imp) / E
        var = jnp.sum(jnp.where(vmask, (imp - mean) ** 2, 0.0)) / (E - 1)
        loss_ref[...] = var / (mean * mean + 1e-10) * jnp.ones_like(loss_ref)


def _expert_body(x_ref, w1_ref, b1_ref, w2_ref, b2_ref, tv_ref, idx_ref,
                 g_ref, out_ref, *, E, BT, D):
    e = pl.program_id(0)
    t = pl.program_id(1)
    sl = pl.ds(t * BT, BT)
    xb = x_ref[sl, :]
    bf = jnp.bfloat16
    h = jnp.dot(xb.astype(bf), w1_ref[0].astype(bf),
                preferred_element_type=jnp.float32) + b1_ref[0]
    h = _gelu(h)
    o = jnp.dot(h.astype(bf), w2_ref[0].astype(bf),
                preferred_element_type=jnp.float32) + b2_ref[0]
    contrib = jnp.where(idx_ref[sl, :] == e, o, 0.0)

    @pl.when(e == 0)
    def _():
        out_ref[sl, :] = contrib

    @pl.when(e != 0)
    def _():
        out_ref[sl, :] += contrib

    @pl.when(e == E - 1)
    def _():
        y = xb + out_ref[sl, :] * tv_ref[sl, :]
        nrm = jnp.sqrt(jnp.sum(y * y, axis=1, keepdims=True))
        y_n = y / jnp.maximum(nrm, 1e-12) * g_ref[...] * (D ** 0.5)
        out_ref[sl, :] = _gelu(y_n)


def kernel(x, gate_W, gate_b, W1, b1, W2, b2, gamma):
    B, N, D = x.shape
    E, _, H = W1.shape
    x_flat = x.reshape(N, D)
    BT = 256
    T = N // BT

    gwp = jnp.zeros((D, LANES), jnp.float32).at[:, :E].set(gate_W)
    gbp = jnp.zeros((1, LANES), jnp.float32).at[0, :E].set(gate_b)

    tv, idx, loss = pl.pallas_call(
        functools.partial(_gate_body, E=E),
        grid=(T,),
        in_specs=[
            pl.BlockSpec((BT, D), lambda t: (t, 0)),
            pl.BlockSpec((D, LANES), lambda t: (0, 0)),
            pl.BlockSpec((1, LANES), lambda t: (0, 0)),
        ],
        out_specs=[
            pl.BlockSpec((BT, 1), lambda t: (t, 0)),
            pl.BlockSpec((BT, 1), lambda t: (t, 0)),
            pl.BlockSpec((1, 1), lambda t: (0, 0)),
        ],
        out_shape=[
            jax.ShapeDtypeStruct((N, 1), jnp.float32),
            jax.ShapeDtypeStruct((N, 1), jnp.int32),
            jax.ShapeDtypeStruct((1, 1), jnp.float32),
        ],
        scratch_shapes=[pltpu.VMEM((1, LANES), jnp.float32)],
    )(x_flat, gwp, gbp)

    out = pl.pallas_call(
        functools.partial(_expert_body, E=E, BT=BT, D=D),
        grid=(E, T),
        in_specs=[
            pl.BlockSpec((N, D), lambda e, t: (0, 0)),
            pl.BlockSpec((1, D, H), lambda e, t: (e, 0, 0)),
            pl.BlockSpec((1, 1, H), lambda e, t: (e, 0, 0)),
            pl.BlockSpec((1, H, D), lambda e, t: (e, 0, 0)),
            pl.BlockSpec((1, 1, D), lambda e, t: (e, 0, 0)),
            pl.BlockSpec((N, 1), lambda e, t: (0, 0)),
            pl.BlockSpec((N, 1), lambda e, t: (0, 0)),
            pl.BlockSpec((1, D), lambda e, t: (0, 0)),
        ],
        out_specs=pl.BlockSpec((N, D), lambda e, t: (0, 0)),
        out_shape=jax.ShapeDtypeStruct((N, D), jnp.float32),
    )(x_flat, W1, b1.reshape(E, 1, H), W2, b2.reshape(E, 1, D), tv, idx,
      gamma.reshape(1, D))

    return out.reshape(B, N, D), loss.reshape(())


# speedup vs baseline: 1.1605x; 1.1387x over previous
import functools

import jax
import jax.numpy as jnp
from jax import lax
from jax.experimental import pallas as pl
from jax.experimental.pallas import tpu as pltpu
from jax.experimental.pallas import tpu_sc as plsc

LANES = 128
SCL = 16
SC_NC = 2
SC_NS = 16
BT = 128
G = 32


def _gelu(v):
    return 0.5 * v * (1.0 + jax.lax.erf(v * (2.0 ** -0.5)))


def _gate_body(x_ref, gw_ref, gb_ref, tv_ref, idx_ref, r_ref, off_ref,
               tof_ref, eof_ref, fst_ref, lst_ref, npn_ref, loss_ref,
               imp_ref, carry_ref, *, E, BTG):
    t = pl.program_id(0)
    logits = jnp.dot(x_ref[...], gw_ref[...], preferred_element_type=jnp.float32)
    logits = logits + gb_ref[...]
    lane = jax.lax.broadcasted_iota(jnp.int32, logits.shape, 1)
    valid = lane < E
    lm = jnp.where(valid, logits, -1e30)
    m = jnp.max(lm, axis=1, keepdims=True)
    idx = jnp.min(jnp.where(lm == m, lane, LANES), axis=1, keepdims=True)
    onehot = (idx == lane).astype(jnp.float32)
    p = jnp.where(valid, jnp.exp(lm - m), 0.0)
    p = p / jnp.sum(p, axis=1, keepdims=True)
    tv = jnp.sum(p * onehot, axis=1, keepdims=True)
    tv_ref[...] = tv
    idx_ref[...] = idx

    @pl.when(t == 0)
    def _():
        imp_ref[...] = jnp.zeros_like(imp_ref)
        carry_ref[...] = jnp.zeros_like(carry_ref)

    imp_ref[...] += jnp.sum(p, axis=0, keepdims=True)

    ir = jax.lax.broadcasted_iota(jnp.int32, (BTG, BTG), 0)
    ic = jax.lax.broadcasted_iota(jnp.int32, (BTG, BTG), 1)
    tri = (ic < ir).astype(jnp.float32)
    rank_blk = jnp.dot(tri, onehot, preferred_element_type=jnp.float32,
                       precision=jax.lax.Precision.HIGHEST)
    r_tok = jnp.sum(rank_blk * onehot, axis=1, keepdims=True)
    c_tok = jnp.sum(carry_ref[...] * onehot, axis=1, keepdims=True)
    r_ref[...] = (r_tok + c_tok).astype(jnp.int32)
    carry_ref[...] += jnp.sum(onehot, axis=0, keepdims=True)

    @pl.when(t == pl.num_programs(0) - 1)
    def _():
        lr = jax.lax.broadcasted_iota(jnp.int32, (1, LANES), 1)
        jr = jax.lax.broadcasted_iota(jnp.int32, (LANES, LANES), 0)
        jc = jax.lax.broadcasted_iota(jnp.int32, (LANES, LANES), 1)

        def rowdot(v, mm):
            return jnp.dot(v, mm, preferred_element_type=jnp.float32,
                           precision=jax.lax.Precision.HIGHEST)

        def transpose_row(v):
            d = jnp.broadcast_to(v, (LANES, LANES)) * (jr == jc)
            return rowdot(d, jnp.ones((LANES, 1), jnp.float32))

        cnt = carry_ref[...]
        tri_up = (jr < jc).astype(jnp.float32)
        off = rowdot(cnt, tri_up)
        off_ref[...] = off.astype(jnp.int32)

        tt = lr // E
        ee = lr % E
        sel_e = (jr == (jc % E)).astype(jnp.float32)
        sel_e1 = (jr == (jc % E) + 1).astype(jnp.float32)
        offe = rowdot(off, sel_e)
        ende = rowdot(off, sel_e1)
        ttf = tt.astype(jnp.float32)
        flag = ((offe < (ttf + 1.0) * BT) & (ende > ttf * BT)
                & (ende > offe)).astype(jnp.float32)
        pos = rowdot(flag, tri_up)
        n = jnp.sum(flag)
        pos_c = transpose_row(pos)
        flag_c = transpose_row(flag)
        stt = ((pos_c == jc.astype(jnp.float32)) & (flag_c > 0.5)
               ).astype(jnp.float32)
        ttv = ttf * flag
        eev = ee.astype(jnp.float32) * flag
        tof = rowdot(ttv, stt)
        eof = rowdot(eev, stt)
        is_last_pos = (pos == n - 1.0).astype(jnp.float32) * flag
        tlast = jnp.sum(ttv * is_last_pos)
        elast = jnp.sum(eev * is_last_pos)
        lrf = lr.astype(jnp.float32)
        tof = jnp.where(lrf <= n - 1.0, tof, tlast)
        eof = jnp.where(lrf <= n - 1.0, eof, elast)
        shl = (jr == jc - 1).astype(jnp.float32)
        shr = (jr == jc + 1).astype(jnp.float32)
        prev = rowdot(tof, shl)
        nxt = rowdot(tof, shr)
        fst = (lr == 0) | (tof != prev)
        lst = (lrf == n - 1.0) | (nxt != tof)
        tof_ref[...] = tof.astype(jnp.int32)
        eof_ref[...] = eof.astype(jnp.int32)
        fst_ref[...] = fst.astype(jnp.int32)
        lst_ref[...] = lst.astype(jnp.int32)
        npn_ref[...] = n.astype(jnp.int32) * jnp.ones_like(npn_ref)

        imp = imp_ref[...]
        vmask = jax.lax.broadcasted_iota(jnp.int32, imp.shape, 1) < E
        mean = jnp.sum(imp) / E
        var = jnp.sum(jnp.where(vmask, (imp - mean) ** 2, 0.0)) / (E - 1)
        loss_ref[...] = var / (mean * mean + 1e-10) * jnp.ones_like(loss_ref)


def _gate(x_flat, gate_W, gate_b):
    N, D = x_flat.shape
    E = gate_W.shape[1]
    BTG = 256
    T = N // BTG
    gwp = jnp.zeros((D, LANES), jnp.float32).at[:, :E].set(gate_W)
    gbp = jnp.zeros((1, LANES), jnp.float32).at[0, :E].set(gate_b)
    return pl.pallas_call(
        functools.partial(_gate_body, E=E, BTG=BTG),
        grid=(T,),
        in_specs=[
            pl.BlockSpec((BTG, D), lambda t: (t, 0)),
            pl.BlockSpec((D, LANES), lambda t: (0, 0)),
            pl.BlockSpec((1, LANES), lambda t: (0, 0)),
        ],
        out_specs=[
            pl.BlockSpec((BTG, 1), lambda t: (t, 0)),
            pl.BlockSpec((BTG, 1), lambda t: (t, 0)),
            pl.BlockSpec((BTG, 1), lambda t: (t, 0)),
            pl.BlockSpec((1, LANES), lambda t: (0, 0)),
            pl.BlockSpec((1, LANES), lambda t: (0, 0)),
            pl.BlockSpec((1, LANES), lambda t: (0, 0)),
            pl.BlockSpec((1, LANES), lambda t: (0, 0)),
            pl.BlockSpec((1, LANES), lambda t: (0, 0)),
            pl.BlockSpec((1, 1), lambda t: (0, 0)),
            pl.BlockSpec((1, 1), lambda t: (0, 0)),
        ],
        out_shape=[
            jax.ShapeDtypeStruct((N, 1), jnp.float32),
            jax.ShapeDtypeStruct((N, 1), jnp.int32),
            jax.ShapeDtypeStruct((N, 1), jnp.int32),
            jax.ShapeDtypeStruct((1, LANES), jnp.int32),
            jax.ShapeDtypeStruct((1, LANES), jnp.int32),
            jax.ShapeDtypeStruct((1, LANES), jnp.int32),
            jax.ShapeDtypeStruct((1, LANES), jnp.int32),
            jax.ShapeDtypeStruct((1, LANES), jnp.int32),
            jax.ShapeDtypeStruct((1, 1), jnp.int32),
            jax.ShapeDtypeStruct((1, 1), jnp.float32),
        ],
        scratch_shapes=[pltpu.VMEM((1, LANES), jnp.float32),
                        pltpu.VMEM((1, LANES), jnp.float32)],
    )(x_flat, gwp, gbp)


def _dispatch_sc(x_flat, idxf, rf, tvf, offsets):
    N, D = x_flat.shape
    NW = SC_NC * SC_NS
    CHUNK = N // NW
    mesh = plsc.VectorSubcoreMesh(core_axis_name="c", subcore_axis_name="s")
    gdn = lax.GatherDimensionNumbers(
        offset_dims=(), collapsed_slice_dims=(0,), start_index_map=(0,))

    @functools.partial(
        pl.kernel, mesh=mesh,
        out_type=[
            jax.ShapeDtypeStruct((N, D), jnp.float32),
            jax.ShapeDtypeStruct((N, LANES), jnp.float32),
            jax.ShapeDtypeStruct((N,), jnp.int32),
        ],
        scratch_types=[
            pltpu.VMEM((SCL,), jnp.int32),
            pltpu.VMEM((CHUNK,), jnp.int32),
            pltpu.VMEM((CHUNK,), jnp.int32),
            pltpu.VMEM((1, CHUNK), jnp.int32),
            pltpu.VMEM((CHUNK,), jnp.float32),
            pltpu.VMEM((CHUNK, D), jnp.float32),
            pltpu.VMEM((CHUNK, LANES), jnp.float32),
            pltpu.SemaphoreType.DMA,
        ],
    )
    def k(x_hbm, idx_hbm, r_hbm, tv_hbm, offs_hbm,
          xs_hbm, tvp_hbm, dest_hbm,
          off_v, idx_v, r_v, dest_v, tv_v, rows_v, tvrow_v, sem):
        wid = lax.axis_index("s") * SC_NC + lax.axis_index("c")
        base = wid * CHUNK
        pltpu.sync_copy(offs_hbm, off_v)
        pltpu.sync_copy(idx_hbm.at[pl.ds(base, CHUNK)], idx_v)
        pltpu.sync_copy(r_hbm.at[pl.ds(base, CHUNK)], r_v)
        pltpu.sync_copy(tv_hbm.at[pl.ds(base, CHUNK)], tv_v)
        pltpu.sync_copy(x_hbm.at[pl.ds(base, CHUNK)], rows_v)
        off_vec = off_v[...]
        lane = lax.iota(jnp.int32, SCL)
        for j in range(CHUNK // SCL):
            sl = pl.ds(j * SCL, SCL)
            ev = idx_v[sl]
            offe = lax.gather(off_vec, ev[:, None], dimension_numbers=gdn,
                              slice_sizes=(1,),
                              mode=lax.GatherScatterMode.PROMISE_IN_BOUNDS)
            dest_v[0, sl] = offe + r_v[sl]
            tvg = tv_v[sl]
            for jj in range(SCL):
                tvrow_v[j * SCL + jj, pl.ds(0, SCL)] = jnp.where(
                    lane == 0, tvg[jj], 0.0)
        pltpu.sync_copy(dest_v.at[0], dest_hbm.at[pl.ds(base, CHUNK)])
        pltpu.async_copy(rows_v, xs_hbm.at[dest_v.at[0]], sem).wait()
        pltpu.async_copy(tvrow_v, tvp_hbm.at[dest_v.at[0]], sem).wait()

    return k(x_flat, idxf, rf, tvf, offsets)


def _ffn_body(tof_ref, eof_ref, fst_ref, lst_ref, np_ref, off_ref,
              xs_ref, w1_ref, b1_ref, w2_ref, b2_ref, tvp_ref, g_ref,
              out_ref, *, D):
    g = pl.program_id(0)

    @pl.when(g < np_ref[0])
    def _():
        t = tof_ref[g]
        e = eof_ref[g]
        xb = xs_ref[...]
        h = _gelu(jnp.dot(xb, w1_ref[0], preferred_element_type=jnp.float32)
                  + b1_ref[0])
        o = jnp.dot(h, w2_ref[0], preferred_element_type=jnp.float32) + b2_ref[0]
        rows = t * BT + jax.lax.broadcasted_iota(jnp.int32, (BT, 1), 0)
        seg = (rows >= off_ref[e]) & (rows < off_ref[e + 1])
        contrib = jnp.where(seg, o, 0.0)

        @pl.when(fst_ref[g] == 1)
        def _():
            out_ref[...] = contrib

        @pl.when(fst_ref[g] == 0)
        def _():
            out_ref[...] += contrib

        @pl.when(lst_ref[g] == 1)
        def _():
            y = xb + out_ref[...] * tvp_ref[:, 0:1]
            nrm = jnp.sqrt(jnp.sum(y * y, axis=1, keepdims=True))
            y_n = y / jnp.maximum(nrm, 1e-12) * g_ref[...] * (D ** 0.5)
            out_ref[...] = _gelu(y_n)


def _grouped_ffn(xs, tvp, W1, b1, W2, b2, gamma, tof, eof, fst, lst, npv, off):
    N, D = xs.shape
    E, _, H = W1.shape
    grid_spec = pltpu.PrefetchScalarGridSpec(
        num_scalar_prefetch=6,
        grid=(G,),
        in_specs=[
            pl.BlockSpec((BT, D), lambda g, tof, eof, fst, lst, npv, off: (tof[g], 0)),
            pl.BlockSpec((1, D, H), lambda g, tof, eof, fst, lst, npv, off: (eof[g], 0, 0)),
            pl.BlockSpec((1, 1, H), lambda g, tof, eof, fst, lst, npv, off: (eof[g], 0, 0)),
            pl.BlockSpec((1, H, D), lambda g, tof, eof, fst, lst, npv, off: (eof[g], 0, 0)),
            pl.BlockSpec((1, 1, D), lambda g, tof, eof, fst, lst, npv, off: (eof[g], 0, 0)),
            pl.BlockSpec((BT, LANES), lambda g, tof, eof, fst, lst, npv, off: (tof[g], 0)),
            pl.BlockSpec((1, D), lambda g, tof, eof, fst, lst, npv, off: (0, 0)),
        ],
        out_specs=pl.BlockSpec((BT, D), lambda g, tof, eof, fst, lst, npv, off: (tof[g], 0)),
    )
    return pl.pallas_call(
        functools.partial(_ffn_body, D=D),
        grid_spec=grid_spec,
        out_shape=jax.ShapeDtypeStruct((N, D), jnp.float32),
    )(tof, eof, fst, lst, npv, off,
      xs, W1, b1.reshape(E, 1, H), W2, b2.reshape(E, 1, D), tvp,
      gamma.reshape(1, D))


def _unsort_sc(y_sorted, dest):
    N, D = y_sorted.shape
    NW = SC_NC * SC_NS
    CHUNK = N // NW
    mesh = plsc.VectorSubcoreMesh(core_axis_name="c", subcore_axis_name="s")

    @functools.partial(
        pl.kernel, mesh=mesh,
        out_type=jax.ShapeDtypeStruct((N, D), jnp.float32),
        scratch_types=[
            pltpu.VMEM((CHUNK,), jnp.int32),
            pltpu.VMEM((CHUNK, D), jnp.float32),
            pltpu.SemaphoreType.DMA,
        ],
    )
    def k(y_hbm, dest_hbm, out_hbm, d_v, rows_v, sem):
        wid = lax.axis_index("s") * SC_NC + lax.axis_index("c")
        base = wid * CHUNK
        pltpu.sync_copy(dest_hbm.at[pl.ds(base, CHUNK)], d_v)
        pltpu.async_copy(y_hbm.at[d_v], rows_v, sem).wait()
        pltpu.sync_copy(rows_v, out_hbm.at[pl.ds(base, CHUNK)])

    return k(y_sorted, dest)


def kernel(x, gate_W, gate_b, W1, b1, W2, b2, gamma):
    B, N, D = x.shape
    x_flat = x.reshape(N, D)

    (tv, idx, r, offp, tofp, eofp, fstp, lstp, npn, loss) = _gate(
        x_flat, gate_W, gate_b)

    xs, tvp, dest = _dispatch_sc(
        x_flat, idx.reshape(N), r.reshape(N), tv.reshape(N), offp[0, :SCL])

    y_sorted = _grouped_ffn(
        xs, tvp, W1, b1, W2, b2, gamma,
        tofp[0, :G], eofp[0, :G], fstp[0, :G], lstp[0, :G],
        npn.reshape(1), offp[0, :SCL])

    out = _unsort_sc(y_sorted, dest)
    return out.reshape(B, N, D), loss.reshape(())


# speedup vs baseline: 1.1778x; 1.0149x over previous
import functools

import jax
import jax.numpy as jnp
from jax import lax
from jax.experimental import pallas as pl
from jax.experimental.pallas import tpu as pltpu
from jax.experimental.pallas import tpu_sc as plsc

LANES = 128
SCL = 16
SC_NC = 2
SC_NS = 16
BT = 128
G = 32


def _gelu(v):
    return 0.5 * v * (1.0 + jax.lax.erf(v * (2.0 ** -0.5)))


def _gate_body(x_ref, gw_ref, gb_ref, tv_ref, idx_ref, r_ref, off_ref,
               tof_ref, eof_ref, fst_ref, lst_ref, npn_ref, loss_ref,
               imp_ref, carry_ref, *, E, BTG):
    t = pl.program_id(0)
    logits = jnp.dot(x_ref[...], gw_ref[...], preferred_element_type=jnp.float32)
    logits = logits + gb_ref[...]
    lane = jax.lax.broadcasted_iota(jnp.int32, logits.shape, 1)
    valid = lane < E
    lm = jnp.where(valid, logits, -1e30)
    m = jnp.max(lm, axis=1, keepdims=True)
    idx = jnp.min(jnp.where(lm == m, lane, LANES), axis=1, keepdims=True)
    onehot = (idx == lane).astype(jnp.float32)
    p = jnp.where(valid, jnp.exp(lm - m), 0.0)
    p = p / jnp.sum(p, axis=1, keepdims=True)
    tv = jnp.sum(p * onehot, axis=1, keepdims=True)
    tv_ref[...] = tv
    idx_ref[...] = idx

    @pl.when(t == 0)
    def _():
        imp_ref[...] = jnp.zeros_like(imp_ref)
        carry_ref[...] = jnp.zeros_like(carry_ref)

    imp_ref[...] += jnp.sum(p, axis=0, keepdims=True)

    ir = jax.lax.broadcasted_iota(jnp.int32, (BTG, BTG), 0)
    ic = jax.lax.broadcasted_iota(jnp.int32, (BTG, BTG), 1)
    tri = (ic < ir).astype(jnp.float32)
    rank_blk = jnp.dot(tri, onehot, preferred_element_type=jnp.float32)
    r_tok = jnp.sum(rank_blk * onehot, axis=1, keepdims=True)
    c_tok = jnp.sum(carry_ref[...] * onehot, axis=1, keepdims=True)
    r_ref[...] = (r_tok + c_tok).astype(jnp.int32)
    carry_ref[...] += jnp.sum(onehot, axis=0, keepdims=True)

    @pl.when(t == pl.num_programs(0) - 1)
    def _():
        lr = jax.lax.broadcasted_iota(jnp.int32, (1, LANES), 1)
        jr = jax.lax.broadcasted_iota(jnp.int32, (LANES, LANES), 0)
        jc = jax.lax.broadcasted_iota(jnp.int32, (LANES, LANES), 1)

        def rowdot(v, mm):
            return jnp.dot(v, mm, preferred_element_type=jnp.float32,
                           precision=jax.lax.Precision.HIGHEST)

        def transpose_row(v):
            d = jnp.broadcast_to(v, (LANES, LANES)) * (jr == jc)
            return rowdot(d, jnp.ones((LANES, 1), jnp.float32))

        cnt = carry_ref[...]
        tri_up = (jr < jc).astype(jnp.float32)
        off = rowdot(cnt, tri_up)
        off_ref[...] = off.astype(jnp.int32)

        tt = lr // E
        ee = lr % E
        sel_e = (jr == (jc % E)).astype(jnp.float32)
        sel_e1 = (jr == (jc % E) + 1).astype(jnp.float32)
        offe = rowdot(off, sel_e)
        ende = rowdot(off, sel_e1)
        ttf = tt.astype(jnp.float32)
        flag = ((offe < (ttf + 1.0) * BT) & (ende > ttf * BT)
                & (ende > offe)).astype(jnp.float32)
        pos = rowdot(flag, tri_up)
        n = jnp.sum(flag)
        pos_c = transpose_row(pos)
        flag_c = transpose_row(flag)
        stt = ((pos_c == jc.astype(jnp.float32)) & (flag_c > 0.5)
               ).astype(jnp.float32)
        ttv = ttf * flag
        eev = ee.astype(jnp.float32) * flag
        tof = rowdot(ttv, stt)
        eof = rowdot(eev, stt)
        is_last_pos = (pos == n - 1.0).astype(jnp.float32) * flag
        tlast = jnp.sum(ttv * is_last_pos)
        elast = jnp.sum(eev * is_last_pos)
        lrf = lr.astype(jnp.float32)
        tof = jnp.where(lrf <= n - 1.0, tof, tlast)
        eof = jnp.where(lrf <= n - 1.0, eof, elast)
        shl = (jr == jc - 1).astype(jnp.float32)
        shr = (jr == jc + 1).astype(jnp.float32)
        prev = rowdot(tof, shl)
        nxt = rowdot(tof, shr)
        fst = (lr == 0) | (tof != prev)
        lst = (lrf == n - 1.0) | (nxt != tof)
        tof_ref[...] = tof.astype(jnp.int32)
        eof_ref[...] = eof.astype(jnp.int32)
        fst_ref[...] = fst.astype(jnp.int32)
        lst_ref[...] = lst.astype(jnp.int32)
        npn_ref[...] = n.astype(jnp.int32) * jnp.ones_like(npn_ref)

        imp = imp_ref[...]
        vmask = jax.lax.broadcasted_iota(jnp.int32, imp.shape, 1) < E
        mean = jnp.sum(imp) / E
        var = jnp.sum(jnp.where(vmask, (imp - mean) ** 2, 0.0)) / (E - 1)
        loss_ref[...] = var / (mean * mean + 1e-10) * jnp.ones_like(loss_ref)


def _gate(x_flat, gate_W, gate_b):
    N, D = x_flat.shape
    E = gate_W.shape[1]
    BTG = 256
    T = N // BTG
    gwp = jnp.zeros((D, LANES), jnp.float32).at[:, :E].set(gate_W)
    gbp = jnp.zeros((1, LANES), jnp.float32).at[0, :E].set(gate_b)
    return pl.pallas_call(
        functools.partial(_gate_body, E=E, BTG=BTG),
        grid=(T,),
        in_specs=[
            pl.BlockSpec((BTG, D), lambda t: (t, 0)),
            pl.BlockSpec((D, LANES), lambda t: (0, 0)),
            pl.BlockSpec((1, LANES), lambda t: (0, 0)),
        ],
        out_specs=[
            pl.BlockSpec((BTG, 1), lambda t: (t, 0)),
            pl.BlockSpec((BTG, 1), lambda t: (t, 0)),
            pl.BlockSpec((BTG, 1), lambda t: (t, 0)),
            pl.BlockSpec((1, LANES), lambda t: (0, 0)),
            pl.BlockSpec((1, LANES), lambda t: (0, 0)),
            pl.BlockSpec((1, LANES), lambda t: (0, 0)),
            pl.BlockSpec((1, LANES), lambda t: (0, 0)),
            pl.BlockSpec((1, LANES), lambda t: (0, 0)),
            pl.BlockSpec((1, 1), lambda t: (0, 0)),
            pl.BlockSpec((1, 1), lambda t: (0, 0)),
        ],
        out_shape=[
            jax.ShapeDtypeStruct((N, 1), jnp.float32),
            jax.ShapeDtypeStruct((N, 1), jnp.int32),
            jax.ShapeDtypeStruct((N, 1), jnp.int32),
            jax.ShapeDtypeStruct((1, LANES), jnp.int32),
            jax.ShapeDtypeStruct((1, LANES), jnp.int32),
            jax.ShapeDtypeStruct((1, LANES), jnp.int32),
            jax.ShapeDtypeStruct((1, LANES), jnp.int32),
            jax.ShapeDtypeStruct((1, LANES), jnp.int32),
            jax.ShapeDtypeStruct((1, 1), jnp.int32),
            jax.ShapeDtypeStruct((1, 1), jnp.float32),
        ],
        scratch_shapes=[pltpu.VMEM((1, LANES), jnp.float32),
                        pltpu.VMEM((1, LANES), jnp.float32)],
    )(x_flat, gwp, gbp)


def _dispatch_sc(x_flat, idxf, rf, tvf, offsets):
    N, D = x_flat.shape
    NW = SC_NC * SC_NS
    CHUNK = N // NW
    mesh = plsc.VectorSubcoreMesh(core_axis_name="c", subcore_axis_name="s")
    gdn = lax.GatherDimensionNumbers(
        offset_dims=(), collapsed_slice_dims=(0,), start_index_map=(0,))

    @functools.partial(
        pl.kernel, mesh=mesh,
        out_type=[
            jax.ShapeDtypeStruct((N, D), jnp.float32),
            jax.ShapeDtypeStruct((N, LANES), jnp.float32),
            jax.ShapeDtypeStruct((N,), jnp.int32),
        ],
        scratch_types=[
            pltpu.VMEM((SCL,), jnp.int32),
            pltpu.VMEM((CHUNK,), jnp.int32),
            pltpu.VMEM((CHUNK,), jnp.int32),
            pltpu.VMEM((1, CHUNK), jnp.int32),
            pltpu.VMEM((CHUNK,), jnp.float32),
            pltpu.VMEM((CHUNK, D), jnp.float32),
            pltpu.VMEM((CHUNK, LANES), jnp.float32),
            pltpu.SemaphoreType.DMA,
        ],
    )
    def k(x_hbm, idx_hbm, r_hbm, tv_hbm, offs_hbm,
          xs_hbm, tvp_hbm, dest_hbm,
          off_v, idx_v, r_v, dest_v, tv_v, rows_v, tvrow_v, sem):
        wid = lax.axis_index("s") * SC_NC + lax.axis_index("c")
        base = wid * CHUNK
        pltpu.sync_copy(offs_hbm, off_v)
        pltpu.sync_copy(idx_hbm.at[pl.ds(base, CHUNK)], idx_v)
        pltpu.sync_copy(r_hbm.at[pl.ds(base, CHUNK)], r_v)
        pltpu.sync_copy(tv_hbm.at[pl.ds(base, CHUNK)], tv_v)
        pltpu.sync_copy(x_hbm.at[pl.ds(base, CHUNK)], rows_v)
        off_vec = off_v[...]
        lane = lax.iota(jnp.int32, SCL)
        for j in range(CHUNK // SCL):
            sl = pl.ds(j * SCL, SCL)
            ev = idx_v[sl]
            offe = lax.gather(off_vec, ev[:, None], dimension_numbers=gdn,
                              slice_sizes=(1,),
                              mode=lax.GatherScatterMode.PROMISE_IN_BOUNDS)
            dest_v[0, sl] = offe + r_v[sl]
            tvg = tv_v[sl]
            for jj in range(SCL):
                tvrow_v[j * SCL + jj, pl.ds(0, SCL)] = jnp.where(
                    lane == 0, tvg[jj], 0.0)
        pltpu.sync_copy(dest_v.at[0], dest_hbm.at[pl.ds(base, CHUNK)])
        pltpu.async_copy(rows_v, xs_hbm.at[dest_v.at[0]], sem).wait()
        pltpu.async_copy(tvrow_v, tvp_hbm.at[dest_v.at[0]], sem).wait()

    return k(x_flat, idxf, rf, tvf, offsets)


def _ffn_body(tof_ref, eof_ref, fst_ref, lst_ref, np_ref, off_ref,
              xs_ref, w1_ref, b1_ref, w2_ref, b2_ref, tvp_ref, g_ref,
              out_ref, *, D):
    g = pl.program_id(0)

    @pl.when(g < np_ref[0])
    def _():
        t = tof_ref[g]
        e = eof_ref[g]
        xb = xs_ref[...]
        h = _gelu(jnp.dot(xb, w1_ref[0], preferred_element_type=jnp.float32)
                  + b1_ref[0])
        o = jnp.dot(h, w2_ref[0], preferred_element_type=jnp.float32) + b2_ref[0]
        rows = t * BT + jax.lax.broadcasted_iota(jnp.int32, (BT, 1), 0)
        seg = (rows >= off_ref[e]) & (rows < off_ref[e + 1])
        contrib = jnp.where(seg, o, 0.0)

        @pl.when(fst_ref[g] == 1)
        def _():
            out_ref[...] = contrib

        @pl.when(fst_ref[g] == 0)
        def _():
            out_ref[...] += contrib

        @pl.when(lst_ref[g] == 1)
        def _():
            y = xb + out_ref[...] * tvp_ref[:, 0:1]
            nrm = jnp.sqrt(jnp.sum(y * y, axis=1, keepdims=True))
            y_n = y / jnp.maximum(nrm, 1e-12) * g_ref[...] * (D ** 0.5)
            out_ref[...] = _gelu(y_n)


def _grouped_ffn(xs, tvp, W1, b1, W2, b2, gamma, tof, eof, fst, lst, npv, off):
    N, D = xs.shape
    E, _, H = W1.shape
    grid_spec = pltpu.PrefetchScalarGridSpec(
        num_scalar_prefetch=6,
        grid=(G,),
        in_specs=[
            pl.BlockSpec((BT, D), lambda g, tof, eof, fst, lst, npv, off: (tof[g], 0)),
            pl.BlockSpec((1, D, H), lambda g, tof, eof, fst, lst, npv, off: (eof[g], 0, 0)),
            pl.BlockSpec((1, 1, H), lambda g, tof, eof, fst, lst, npv, off: (eof[g], 0, 0)),
            pl.BlockSpec((1, H, D), lambda g, tof, eof, fst, lst, npv, off: (eof[g], 0, 0)),
            pl.BlockSpec((1, 1, D), lambda g, tof, eof, fst, lst, npv, off: (eof[g], 0, 0)),
            pl.BlockSpec((BT, LANES), lambda g, tof, eof, fst, lst, npv, off: (tof[g], 0)),
            pl.BlockSpec((1, D), lambda g, tof, eof, fst, lst, npv, off: (0, 0)),
        ],
        out_specs=pl.BlockSpec((BT, D), lambda g, tof, eof, fst, lst, npv, off: (tof[g], 0)),
    )
    return pl.pallas_call(
        functools.partial(_ffn_body, D=D),
        grid_spec=grid_spec,
        out_shape=jax.ShapeDtypeStruct((N, D), jnp.float32),
    )(tof, eof, fst, lst, npv, off,
      xs, W1, b1.reshape(E, 1, H), W2, b2.reshape(E, 1, D), tvp,
      gamma.reshape(1, D))


def _unsort_sc(y_sorted, dest):
    N, D = y_sorted.shape
    NW = SC_NC * SC_NS
    CHUNK = N // NW
    mesh = plsc.VectorSubcoreMesh(core_axis_name="c", subcore_axis_name="s")

    @functools.partial(
        pl.kernel, mesh=mesh,
        out_type=jax.ShapeDtypeStruct((N, D), jnp.float32),
        scratch_types=[
            pltpu.VMEM((CHUNK,), jnp.int32),
            pltpu.VMEM((CHUNK, D), jnp.float32),
            pltpu.SemaphoreType.DMA,
        ],
    )
    def k(y_hbm, dest_hbm, out_hbm, d_v, rows_v, sem):
        wid = lax.axis_index("s") * SC_NC + lax.axis_index("c")
        base = wid * CHUNK
        pltpu.sync_copy(dest_hbm.at[pl.ds(base, CHUNK)], d_v)
        pltpu.async_copy(y_hbm.at[d_v], rows_v, sem).wait()
        pltpu.sync_copy(rows_v, out_hbm.at[pl.ds(base, CHUNK)])

    return k(y_sorted, dest)


def kernel(x, gate_W, gate_b, W1, b1, W2, b2, gamma):
    B, N, D = x.shape
    x_flat = x.reshape(N, D)

    (tv, idx, r, offp, tofp, eofp, fstp, lstp, npn, loss) = _gate(
        x_flat, gate_W, gate_b)

    xs, tvp, dest = _dispatch_sc(
        x_flat, idx.reshape(N), r.reshape(N), tv.reshape(N), offp[0, :SCL])

    y_sorted = _grouped_ffn(
        xs, tvp, W1, b1, W2, b2, gamma,
        tofp[0, :G], eofp[0, :G], fstp[0, :G], lstp[0, :G],
        npn.reshape(1), offp[0, :SCL])

    out = _unsort_sc(y_sorted, dest)
    return out.reshape(B, N, D), loss.reshape(())
